# baseline (speedup 1.0000x reference)
import functools

import jax
import jax.numpy as jnp
from jax import lax
from jax.experimental import pallas as pl
from jax.experimental.pallas import tpu as pltpu
from jax.experimental.pallas import tpu_sc as plsc

N = 10000
E = 320000
D = 128
G = 128
NUM_LAYERS = 4

NC = 2
NS = 16
NW = NC * NS
K = 128
EPW = 10240
NB = EPW // K
EREAL = E // NW
NPAD = 10240
RP = NPAD // NS

R = 1000
NBLK = N // R


def _make_agg():
    mesh = plsc.VectorSubcoreMesh(core_axis_name="c", subcore_axis_name="s")

    @functools.partial(
        pl.kernel,
        mesh=mesh,
        out_type=jax.ShapeDtypeStruct((NC, NPAD, D), jnp.float32),
        scratch_types=[
            pltpu.VMEM((K,), jnp.int32),
            pltpu.VMEM((K,), jnp.int32),
            pltpu.VMEM((K,), jnp.int32),
            pltpu.VMEM((K,), jnp.int32),
            pltpu.VMEM((K, D), jnp.float32),
            pltpu.VMEM((K, D), jnp.float32),
            pltpu.VMEM_SHARED((NPAD, D), jnp.float32),
            pltpu.SemaphoreType.DMA,
            pltpu.SemaphoreType.DMA,
            pltpu.SemaphoreType.DMA,
            pltpu.SemaphoreType.DMA,
        ],
    )
    def agg(h_hbm, src_hbm, dst_hbm, zrows_hbm, out_hbm, sidx0, didx0, sidx1,
            didx1, rows0, rows1, shared, gsem0, gsem1, ssem0, ssem1):
        c = lax.axis_index("c")
        s = lax.axis_index("s")
        wid = s * NC + c
        r0 = s * RP
        pltpu.sync_copy(zrows_hbm, shared.at[pl.ds(r0, RP)])
        plsc.subcore_barrier()
        e0 = wid * EPW

        bufs = ((sidx0, didx0, rows0, gsem0, ssem0),
                (sidx1, didx1, rows1, gsem1, ssem1))

        def step(b, mine, other):
            sidx, didx, rows, gsem, ssem = mine
            _, odidx, orows, ogsem, ossem = other
            eb = e0 + b * K

            @pl.when(b >= 2)
            def _():
                pltpu.make_async_copy(rows, shared.at[didx], ssem).wait()

            pltpu.sync_copy(src_hbm.at[pl.ds(eb, K)], sidx)
            pltpu.sync_copy(dst_hbm.at[pl.ds(eb, K)], didx)
            pltpu.async_copy(h_hbm.at[sidx], rows, gsem)

            @pl.when(b >= 1)
            def _():
                pltpu.make_async_copy(h_hbm.at[sidx], orows, ogsem).wait()
                pltpu.async_copy(orows, shared.at[odidx], ossem, add=True)

        def body(b, carry):
            @pl.when(lax.rem(b, 2) == 0)
            def _():
                step(b, bufs[0], bufs[1])

            @pl.when(lax.rem(b, 2) == 1)
            def _():
                step(b, bufs[1], bufs[0])

            return carry

        lax.fori_loop(0, NB, body, 0)

        lsidx, ldidx, lrows, lgsem, lssem = bufs[(NB - 1) % 2]
        _, pdidx, prows, _, pssem = bufs[NB % 2]
        pltpu.make_async_copy(h_hbm.at[lsidx], lrows, lgsem).wait()
        pltpu.async_copy(lrows, shared.at[ldidx], lssem, add=True)
        pltpu.make_async_copy(prows, shared.at[pdidx], pssem).wait()
        pltpu.make_async_copy(lrows, shared.at[ldidx], lssem).wait()

        plsc.subcore_barrier()
        pltpu.sync_copy(shared.at[pl.ds(r0, RP)], out_hbm.at[c, pl.ds(r0, RP)])

    return agg


_agg_cache = []


def _agg(h, src, dst, zrows):
    if not _agg_cache:
        _agg_cache.append(_make_agg())
    return _agg_cache[0](h, src, dst, zrows)


def _k1(h_ref, a_ref, w1_ref, b1_ref, w2_ref, b2_ref, u_ref, st_ref, acc_ref):
    z = h_ref[...] + a_ref[0] + a_ref[1]
    t = jnp.dot(z, w1_ref[...], preferred_element_type=jnp.float32) + b1_ref[...]
    t = jnp.maximum(t, 0.0)
    u = jnp.dot(t, w2_ref[...], preferred_element_type=jnp.float32) + b2_ref[...]
    u = jnp.maximum(u, 0.0)
    u_ref[...] = u
    ssum = jnp.sum(u, axis=0, keepdims=True)
    ssq = jnp.sum(u * u, axis=0, keepdims=True)
    stats = jnp.concatenate([ssum, ssq], axis=0)

    @pl.when(pl.program_id(0) == 0)
    def _():
        acc_ref[...] = jnp.zeros_like(acc_ref)

    acc_ref[...] += stats

    @pl.when(pl.program_id(0) == pl.num_programs(0) - 1)
    def _():
        st_ref[...] = acc_ref[...]


def _mlp_stats(h, agg, w1, b1, w2, b2):
    return pl.pallas_call(
        _k1,
        grid=(NBLK,),
        in_specs=[
            pl.BlockSpec((R, D), lambda i: (i, 0)),
            pl.BlockSpec((NC, R, D), lambda i: (0, i, 0)),
            pl.BlockSpec((D, D), lambda i: (0, 0)),
            pl.BlockSpec((1, D), lambda i: (0, 0)),
            pl.BlockSpec((D, D), lambda i: (0, 0)),
            pl.BlockSpec((1, D), lambda i: (0, 0)),
        ],
        out_specs=[
            pl.BlockSpec((R, D), lambda i: (i, 0)),
            pl.BlockSpec((2, D), lambda i: (0, 0)),
        ],
        out_shape=[
            jax.ShapeDtypeStruct((N, D), jnp.float32),
            jax.ShapeDtypeStruct((2, D), jnp.float32),
        ],
        scratch_shapes=[pltpu.VMEM((2, D), jnp.float32)],
    )(h, agg, w1, b1, w2, b2)


def _k2(u_ref, st_ref, g_ref, bt_ref, ids_ref, h_ref, p_ref, acc_ref):
    mean = st_ref[0:1] * (1.0 / N)
    var = st_ref[1:2] * (1.0 / N) - mean * mean
    inv = g_ref[...] * lax.rsqrt(var + 1e-5)
    h = (u_ref[...] - mean) * inv + bt_ref[...]
    h_ref[...] = h
    oh = (ids_ref[...] == lax.broadcasted_iota(jnp.int32, (1, G), 1))
    oh = oh.astype(jnp.float32)
    p = lax.dot_general(oh, h, (((0,), (0,)), ((), ())),
                        precision=lax.Precision.HIGHEST,
                        preferred_element_type=jnp.float32)

    @pl.when(pl.program_id(0) == 0)
    def _():
        acc_ref[...] = jnp.zeros_like(acc_ref)

    acc_ref[...] += p

    @pl.when(pl.program_id(0) == pl.num_programs(0) - 1)
    def _():
        p_ref[...] = acc_ref[...]


def _bn_pool(u, st, gam, bet, ids):
    return pl.pallas_call(
        _k2,
        grid=(NBLK,),
        in_specs=[
            pl.BlockSpec((R, D), lambda i: (i, 0)),
            pl.BlockSpec((2, D), lambda i: (0, 0)),
            pl.BlockSpec((1, D), lambda i: (0, 0)),
            pl.BlockSpec((1, D), lambda i: (0, 0)),
            pl.BlockSpec((R, 1), lambda i: (i, 0)),
        ],
        out_specs=[
            pl.BlockSpec((R, D), lambda i: (i, 0)),
            pl.BlockSpec((G, D), lambda i: (0, 0)),
        ],
        out_shape=[
            jax.ShapeDtypeStruct((N, D), jnp.float32),
            jax.ShapeDtypeStruct((G, D), jnp.float32),
        ],
        scratch_shapes=[pltpu.VMEM((G, D), jnp.float32)],
    )(u, st, gam, bet, ids)


def _lrelu(x):
    return jnp.where(x >= 0, x, 0.01 * x)


def _kheads(p_ref, pw1, pb1, pw2, pb2, gw1, gb1, gw2, gb2, gw3, gb3, gws, gbs,
            cw1, cb1, cw2, cb2, cws, cbs, z_ref, g_ref):
    y = jnp.concatenate([p_ref[0], p_ref[1], p_ref[2], p_ref[3]], axis=1)

    def mm(a, w, b):
        return jnp.dot(a, w[...], preferred_element_type=jnp.float32) + b[...]

    y2 = mm(_lrelu(mm(y, pw1, pb1)), pw2, pb2)
    h1 = jnp.maximum(mm(y2, gw1, gb1), 0.0)
    h2 = jnp.maximum(mm(h1, gw2, gb2), 0.0)
    h3 = jnp.maximum(mm(h2, gw3, gb3), 0.0)
    g = h3 + mm(y2, gws, gbs)
    hb = _lrelu(mm(_lrelu(mm(g, cw1, cb1)), cw2, cb2))
    z = hb + mm(g, cws, cbs)
    z_ref[...] = z
    g_ref[...] = g


def _heads(pooled, pp, gd, cp):
    emb = NUM_LAYERS * D

    def r(v):
        return v.reshape(1, -1)

    return pl.pallas_call(
        _kheads,
        out_shape=[
            jax.ShapeDtypeStruct((G, cp['Ws'].shape[1]), jnp.float32),
            jax.ShapeDtypeStruct((G, emb), jnp.float32),
        ],
    )(pooled, pp['W1'], r(pp['b1']), pp['W2'], r(pp['b2']),
      gd['W1'], r(gd['b1']), gd['W2'], r(gd['b2']), gd['W3'], r(gd['b3']),
      gd['Ws'], r(gd['bs']),
      cp['W1'], r(cp['b1']), cp['W2'], r(cp['b2']), cp['Ws'], r(cp['bs']))


def kernel(x, edge_index, batch, num_graphs, params):
    npad = EPW - EREAL
    src = jnp.concatenate(
        [edge_index[0].reshape(NW, EREAL),
         jnp.zeros((NW, npad), jnp.int32)], axis=1).reshape(-1)
    spare = NPAD - N
    pad_dst = (N + (jnp.arange(NW, dtype=jnp.int32)[:, None] * 97
                    + jnp.arange(npad, dtype=jnp.int32)[None, :]) % spare)
    dst = jnp.concatenate(
        [edge_index[1].reshape(NW, EREAL), pad_dst], axis=1).reshape(-1)
    zrows = jnp.zeros((RP, D), jnp.float32)
    ids = batch.reshape(N, 1)

    h = x
    pooled = []
    for i in range(NUM_LAYERS):
        p = params['gin%d' % i]
        agg = _agg(h, src, dst, zrows)
        u, st = _mlp_stats(h, agg, p['W1'], p['b1'].reshape(1, -1),
                           p['W2'], p['b2'].reshape(1, -1))
        h, pool = _bn_pool(u, st, p['bn_g'].reshape(1, -1),
                           p['bn_b'].reshape(1, -1), ids)
        pooled.append(pool)

    pooled = jnp.stack(pooled)
    z, g = _heads(pooled, params['proj'], params['global_d'],
                  params['cluster'])
    return (z, g)


# speedup vs baseline: 3.0587x; 3.0587x over previous
import functools

import jax
import jax.numpy as jnp
from jax import lax
from jax.experimental import pallas as pl
from jax.experimental.pallas import tpu as pltpu
from jax.experimental.pallas import tpu_sc as plsc

N = 10000
E = 320000
D = 128
G = 128
NUM_LAYERS = 4

NC = 2
NS = 16
NW = NC * NS
K = 80
EPW = E // NW
NB = EPW // K
NPAD = 10240
RP = NPAD // NS

R = 1000
NBLK = N // R


def _make_agg():
    mesh = plsc.VectorSubcoreMesh(core_axis_name="c", subcore_axis_name="s")

    @functools.partial(
        pl.kernel,
        mesh=mesh,
        out_type=jax.ShapeDtypeStruct((NC, NPAD, D), jnp.float32),
        scratch_types=(
            [pltpu.VMEM((K,), jnp.int32)] * 8
            + [pltpu.VMEM((K, D), jnp.float32)] * 2
            + [pltpu.VMEM_SHARED((NPAD, D), jnp.float32)]
            + [pltpu.SemaphoreType.DMA] * 8
        ),
    )
    def agg(h_hbm, src_hbm, dst_hbm, zrows_hbm, out_hbm,
            si0, si1, si2, si3, di0, di1, di2, di3, rows0, rows1, shared,
            is0, is1, is2, is3, gsem0, gsem1, ssem0, ssem1):
        c = lax.axis_index("c")
        s = lax.axis_index("s")
        wid = s * NC + c
        r0 = s * RP
        pltpu.sync_copy(zrows_hbm, shared.at[pl.ds(r0, RP)])
        e0 = wid * EPW

        sidx = (si0, si1, si2, si3)
        didx = (di0, di1, di2, di3)
        isem = (is0, is1, is2, is3)
        rows = (rows0, rows1)
        gsem = (gsem0, gsem1)
        ssem = (ssem0, ssem1)

        def pref(b, q):
            eb = e0 + b * K
            pltpu.async_copy(src_hbm.at[pl.ds(eb, K)], sidx[q], isem[q])
            pltpu.async_copy(dst_hbm.at[pl.ds(eb, K)], didx[q], isem[q])

        pref(0, 0)
        pref(1, 1)
        plsc.subcore_barrier()

        def step(b, q):
            p = q % 2
            o = 1 - p
            qp = (q + 3) % 4
            @pl.when(b >= 2)
            def _():
                pltpu.make_async_copy(rows[p], shared.at[didx[q]],
                                      ssem[p]).wait()

            pltpu.make_async_copy(src_hbm.at[pl.ds(e0, K)], sidx[q],
                                  isem[q]).wait()
            pltpu.make_async_copy(dst_hbm.at[pl.ds(e0, K)], didx[q],
                                  isem[q]).wait()
            pltpu.async_copy(h_hbm.at[sidx[q]], rows[p], gsem[p])

            @pl.when(b + 2 < NB)
            def _():
                pref(b + 2, (q + 2) % 4)

            @pl.when(b >= 1)
            def _():
                pltpu.make_async_copy(h_hbm.at[sidx[qp]], rows[o],
                                      gsem[o]).wait()
                pltpu.async_copy(rows[o], shared.at[didx[qp]], ssem[o],
                                 add=True)

        def body(b, carry):
            for q in range(4):
                @pl.when(lax.rem(b, 4) == q)
                def _():
                    step(b, q)

            return carry

        lax.fori_loop(0, NB, body, 0)

        lq = (NB - 1) % 4
        lp = lq % 2
        pltpu.make_async_copy(h_hbm.at[sidx[lq]], rows[lp], gsem[lp]).wait()
        pltpu.async_copy(rows[lp], shared.at[didx[lq]], ssem[lp], add=True)
        pltpu.make_async_copy(rows[1 - lp], shared.at[didx[(lq + 3) % 4]],
                              ssem[1 - lp]).wait()
        pltpu.make_async_copy(rows[lp], shared.at[didx[lq]], ssem[lp]).wait()

        plsc.subcore_barrier()
        pltpu.sync_copy(shared.at[pl.ds(r0, RP)], out_hbm.at[c, pl.ds(r0, RP)])

    return agg


_agg_cache = []


def _agg(h, src, dst, zrows):
    if not _agg_cache:
        _agg_cache.append(_make_agg())
    return _agg_cache[0](h, src, dst, zrows)


def _k1(h_ref, a_ref, w1_ref, b1_ref, w2_ref, b2_ref, u_ref, st_ref, acc_ref):
    z = h_ref[...] + a_ref[0] + a_ref[1]
    t = jnp.dot(z, w1_ref[...], preferred_element_type=jnp.float32) + b1_ref[...]
    t = jnp.maximum(t, 0.0)
    u = jnp.dot(t, w2_ref[...], preferred_element_type=jnp.float32) + b2_ref[...]
    u = jnp.maximum(u, 0.0)
    u_ref[...] = u
    ssum = jnp.sum(u, axis=0, keepdims=True)
    ssq = jnp.sum(u * u, axis=0, keepdims=True)
    stats = jnp.concatenate([ssum, ssq], axis=0)

    @pl.when(pl.program_id(0) == 0)
    def _():
        acc_ref[...] = jnp.zeros_like(acc_ref)

    acc_ref[...] += stats

    @pl.when(pl.program_id(0) == pl.num_programs(0) - 1)
    def _():
        st_ref[...] = acc_ref[...]


def _mlp_stats(h, agg, w1, b1, w2, b2):
    return pl.pallas_call(
        _k1,
        grid=(NBLK,),
        in_specs=[
            pl.BlockSpec((R, D), lambda i: (i, 0)),
            pl.BlockSpec((NC, R, D), lambda i: (0, i, 0)),
            pl.BlockSpec((D, D), lambda i: (0, 0)),
            pl.BlockSpec((1, D), lambda i: (0, 0)),
            pl.BlockSpec((D, D), lambda i: (0, 0)),
            pl.BlockSpec((1, D), lambda i: (0, 0)),
        ],
        out_specs=[
            pl.BlockSpec((R, D), lambda i: (i, 0)),
            pl.BlockSpec((2, D), lambda i: (0, 0)),
        ],
        out_shape=[
            jax.ShapeDtypeStruct((N, D), jnp.float32),
            jax.ShapeDtypeStruct((2, D), jnp.float32),
        ],
        scratch_shapes=[pltpu.VMEM((2, D), jnp.float32)],
    )(h, agg, w1, b1, w2, b2)


def _k2(u_ref, st_ref, g_ref, bt_ref, ids_ref, h_ref, p_ref, acc_ref):
    mean = st_ref[0:1] * (1.0 / N)
    var = st_ref[1:2] * (1.0 / N) - mean * mean
    inv = g_ref[...] * lax.rsqrt(var + 1e-5)
    h = (u_ref[...] - mean) * inv + bt_ref[...]
    h_ref[...] = h
    oh = (ids_ref[...] == lax.broadcasted_iota(jnp.int32, (1, G), 1))
    oh = oh.astype(jnp.float32)
    p = lax.dot_general(oh, h, (((0,), (0,)), ((), ())),
                        precision=lax.Precision.HIGHEST,
                        preferred_element_type=jnp.float32)

    @pl.when(pl.program_id(0) == 0)
    def _():
        acc_ref[...] = jnp.zeros_like(acc_ref)

    acc_ref[...] += p

    @pl.when(pl.program_id(0) == pl.num_programs(0) - 1)
    def _():
        p_ref[...] = acc_ref[...]


def _bn_pool(u, st, gam, bet, ids):
    return pl.pallas_call(
        _k2,
        grid=(NBLK,),
        in_specs=[
            pl.BlockSpec((R, D), lambda i: (i, 0)),
            pl.BlockSpec((2, D), lambda i: (0, 0)),
            pl.BlockSpec((1, D), lambda i: (0, 0)),
            pl.BlockSpec((1, D), lambda i: (0, 0)),
            pl.BlockSpec((R, 1), lambda i: (i, 0)),
        ],
        out_specs=[
            pl.BlockSpec((R, D), lambda i: (i, 0)),
            pl.BlockSpec((G, D), lambda i: (0, 0)),
        ],
        out_shape=[
            jax.ShapeDtypeStruct((N, D), jnp.float32),
            jax.ShapeDtypeStruct((G, D), jnp.float32),
        ],
        scratch_shapes=[pltpu.VMEM((G, D), jnp.float32)],
    )(u, st, gam, bet, ids)


def _lrelu(x):
    return jnp.where(x >= 0, x, 0.01 * x)


def _kheads(p_ref, pw1, pb1, pw2, pb2, gw1, gb1, gw2, gb2, gw3, gb3, gws, gbs,
            cw1, cb1, cw2, cb2, cws, cbs, z_ref, g_ref):
    y = jnp.concatenate([p_ref[0], p_ref[1], p_ref[2], p_ref[3]], axis=1)

    def mm(a, w, b):
        return jnp.dot(a, w[...], preferred_element_type=jnp.float32) + b[...]

    y2 = mm(_lrelu(mm(y, pw1, pb1)), pw2, pb2)
    h1 = jnp.maximum(mm(y2, gw1, gb1), 0.0)
    h2 = jnp.maximum(mm(h1, gw2, gb2), 0.0)
    h3 = jnp.maximum(mm(h2, gw3, gb3), 0.0)
    g = h3 + mm(y2, gws, gbs)
    hb = _lrelu(mm(_lrelu(mm(g, cw1, cb1)), cw2, cb2))
    z = hb + mm(g, cws, cbs)
    z_ref[...] = z
    g_ref[...] = g


def _heads(pooled, pp, gd, cp):
    emb = NUM_LAYERS * D

    def r(v):
        return v.reshape(1, -1)

    return pl.pallas_call(
        _kheads,
        out_shape=[
            jax.ShapeDtypeStruct((G, cp['Ws'].shape[1]), jnp.float32),
            jax.ShapeDtypeStruct((G, emb), jnp.float32),
        ],
    )(pooled, pp['W1'], r(pp['b1']), pp['W2'], r(pp['b2']),
      gd['W1'], r(gd['b1']), gd['W2'], r(gd['b2']), gd['W3'], r(gd['b3']),
      gd['Ws'], r(gd['bs']),
      cp['W1'], r(cp['b1']), cp['W2'], r(cp['b2']), cp['Ws'], r(cp['bs']))


def kernel(x, edge_index, batch, num_graphs, params):
    src = edge_index[0]
    dst = edge_index[1]
    zrows = jnp.zeros((RP, D), jnp.float32)
    ids = batch.reshape(N, 1)

    h = x
    pooled = []
    for i in range(NUM_LAYERS):
        p = params['gin%d' % i]
        agg = _agg(h, src, dst, zrows)
        u, st = _mlp_stats(h, agg, p['W1'], p['b1'].reshape(1, -1),
                           p['W2'], p['b2'].reshape(1, -1))
        h, pool = _bn_pool(u, st, p['bn_g'].reshape(1, -1),
                           p['bn_b'].reshape(1, -1), ids)
        pooled.append(pool)

    pooled = jnp.stack(pooled)
    z, g = _heads(pooled, params['proj'], params['global_d'],
                  params['cluster'])
    return (z, g)


# speedup vs baseline: 3.4610x; 1.1315x over previous
import functools

import jax
import jax.numpy as jnp
from jax import lax
from jax.experimental import pallas as pl
from jax.experimental.pallas import tpu as pltpu
from jax.experimental.pallas import tpu_sc as plsc

N = 10000
E = 320000
D = 128
G = 128
NUM_LAYERS = 4

NC = 2
NS = 16
NW = NC * NS
K = 80
EPW = E // NW
NB = EPW // K
NPAD = 10240
RP = NPAD // NS

R = 1000
NBLK = N // R


def _make_agg():
    mesh = plsc.VectorSubcoreMesh(core_axis_name="c", subcore_axis_name="s")

    NI = 6
    NR = 4

    @functools.partial(
        pl.kernel,
        mesh=mesh,
        out_type=jax.ShapeDtypeStruct((NC, NPAD, D), jnp.float32),
        scratch_types=(
            [pltpu.VMEM((K,), jnp.int32)] * (2 * NI)
            + [pltpu.VMEM((K, D), jnp.float32)] * NR
            + [pltpu.VMEM_SHARED((NPAD, D), jnp.float32)]
            + [pltpu.SemaphoreType.DMA] * (NI + 2 * NR)
        ),
    )
    def agg(h_hbm, src_hbm, dst_hbm, zrows_hbm, out_hbm, *scr):
        sidx = scr[0:NI]
        didx = scr[NI:2 * NI]
        rows = scr[2 * NI:2 * NI + NR]
        shared = scr[2 * NI + NR]
        isem = scr[2 * NI + NR + 1:2 * NI + NR + 1 + NI]
        gsem = scr[2 * NI + NR + 1 + NI:2 * NI + NR + 1 + NI + NR]
        ssem = scr[2 * NI + NR + 1 + NI + NR:]
        c = lax.axis_index("c")
        s = lax.axis_index("s")
        wid = s * NC + c
        r0 = s * RP
        pltpu.sync_copy(zrows_hbm, shared.at[pl.ds(r0, RP)])
        e0 = wid * EPW

        def pref(b, q):
            eb = e0 + b * K
            pltpu.async_copy(src_hbm.at[pl.ds(eb, K)], sidx[q], isem[q])
            pltpu.async_copy(dst_hbm.at[pl.ds(eb, K)], didx[q], isem[q])

        def wait_idx(q):
            pltpu.make_async_copy(src_hbm.at[pl.ds(e0, K)], sidx[q],
                                  isem[q]).wait()
            pltpu.make_async_copy(dst_hbm.at[pl.ds(e0, K)], didx[q],
                                  isem[q]).wait()

        def wait_gather(q6, q4):
            pltpu.make_async_copy(h_hbm.at[sidx[q6]], rows[q4],
                                  gsem[q4]).wait()

        def start_scatter(q6, q4):
            pltpu.async_copy(rows[q4], shared.at[didx[q6]], ssem[q4],
                             add=True)

        def wait_scatter(q6, q4):
            pltpu.make_async_copy(rows[q4], shared.at[didx[q6]],
                                  ssem[q4]).wait()

        pref(0, 0)
        pref(1, 1)
        plsc.subcore_barrier()

        def step(b, q6, q4):
            @pl.when(b >= 4)
            def _():
                wait_scatter((q6 + 2) % NI, q4)

            wait_idx(q6)
            pltpu.async_copy(h_hbm.at[sidx[q6]], rows[q4], gsem[q4])

            @pl.when(b + 2 < NB)
            def _():
                pref(b + 2, (q6 + 2) % NI)

            @pl.when(b >= 2)
            def _():
                wait_gather((q6 + 4) % NI, (q4 + 2) % NR)
                start_scatter((q6 + 4) % NI, (q4 + 2) % NR)

        def body(b, carry):
            for r in range(12):
                @pl.when(lax.rem(b, 12) == r)
                def _():
                    step(b, r % NI, r % NR)

            return carry

        lax.fori_loop(0, NB, body, 0)

        for bb in (NB - 2, NB - 1):
            wait_gather(bb % NI, bb % NR)
            start_scatter(bb % NI, bb % NR)
        for bb in (NB - 4, NB - 3, NB - 2, NB - 1):
            wait_scatter(bb % NI, bb % NR)

        plsc.subcore_barrier()
        pltpu.sync_copy(shared.at[pl.ds(r0, RP)], out_hbm.at[c, pl.ds(r0, RP)])

    return agg


_agg_cache = []


def _agg(h, src, dst, zrows):
    if not _agg_cache:
        _agg_cache.append(_make_agg())
    return _agg_cache[0](h, src, dst, zrows)


def _k1(h_ref, a_ref, w1_ref, b1_ref, w2_ref, b2_ref, u_ref, st_ref, acc_ref):
    z = h_ref[...] + a_ref[0] + a_ref[1]
    t = jnp.dot(z, w1_ref[...], preferred_element_type=jnp.float32) + b1_ref[...]
    t = jnp.maximum(t, 0.0)
    u = jnp.dot(t, w2_ref[...], preferred_element_type=jnp.float32) + b2_ref[...]
    u = jnp.maximum(u, 0.0)
    u_ref[...] = u
    ssum = jnp.sum(u, axis=0, keepdims=True)
    ssq = jnp.sum(u * u, axis=0, keepdims=True)
    stats = jnp.concatenate([ssum, ssq], axis=0)

    @pl.when(pl.program_id(0) == 0)
    def _():
        acc_ref[...] = jnp.zeros_like(acc_ref)

    acc_ref[...] += stats

    @pl.when(pl.program_id(0) == pl.num_programs(0) - 1)
    def _():
        st_ref[...] = acc_ref[...]


def _mlp_stats(h, agg, w1, b1, w2, b2):
    return pl.pallas_call(
        _k1,
        grid=(NBLK,),
        in_specs=[
            pl.BlockSpec((R, D), lambda i: (i, 0)),
            pl.BlockSpec((NC, R, D), lambda i: (0, i, 0)),
            pl.BlockSpec((D, D), lambda i: (0, 0)),
            pl.BlockSpec((1, D), lambda i: (0, 0)),
            pl.BlockSpec((D, D), lambda i: (0, 0)),
            pl.BlockSpec((1, D), lambda i: (0, 0)),
        ],
        out_specs=[
            pl.BlockSpec((R, D), lambda i: (i, 0)),
            pl.BlockSpec((2, D), lambda i: (0, 0)),
        ],
        out_shape=[
            jax.ShapeDtypeStruct((N, D), jnp.float32),
            jax.ShapeDtypeStruct((2, D), jnp.float32),
        ],
        scratch_shapes=[pltpu.VMEM((2, D), jnp.float32)],
    )(h, agg, w1, b1, w2, b2)


def _k2(u_ref, st_ref, g_ref, bt_ref, ids_ref, h_ref, p_ref, acc_ref):
    mean = st_ref[0:1] * (1.0 / N)
    var = st_ref[1:2] * (1.0 / N) - mean * mean
    inv = g_ref[...] * lax.rsqrt(var + 1e-5)
    h = (u_ref[...] - mean) * inv + bt_ref[...]
    h_ref[...] = h
    oh = (ids_ref[...] == lax.broadcasted_iota(jnp.int32, (1, G), 1))
    oh = oh.astype(jnp.float32)
    p = lax.dot_general(oh, h, (((0,), (0,)), ((), ())),
                        precision=lax.Precision.HIGHEST,
                        preferred_element_type=jnp.float32)

    @pl.when(pl.program_id(0) == 0)
    def _():
        acc_ref[...] = jnp.zeros_like(acc_ref)

    acc_ref[...] += p

    @pl.when(pl.program_id(0) == pl.num_programs(0) - 1)
    def _():
        p_ref[...] = acc_ref[...]


def _bn_pool(u, st, gam, bet, ids):
    return pl.pallas_call(
        _k2,
        grid=(NBLK,),
        in_specs=[
            pl.BlockSpec((R, D), lambda i: (i, 0)),
            pl.BlockSpec((2, D), lambda i: (0, 0)),
            pl.BlockSpec((1, D), lambda i: (0, 0)),
            pl.BlockSpec((1, D), lambda i: (0, 0)),
            pl.BlockSpec((R, 1), lambda i: (i, 0)),
        ],
        out_specs=[
            pl.BlockSpec((R, D), lambda i: (i, 0)),
            pl.BlockSpec((G, D), lambda i: (0, 0)),
        ],
        out_shape=[
            jax.ShapeDtypeStruct((N, D), jnp.float32),
            jax.ShapeDtypeStruct((G, D), jnp.float32),
        ],
        scratch_shapes=[pltpu.VMEM((G, D), jnp.float32)],
    )(u, st, gam, bet, ids)


def _lrelu(x):
    return jnp.where(x >= 0, x, 0.01 * x)


def _kheads(p_ref, pw1, pb1, pw2, pb2, gw1, gb1, gw2, gb2, gw3, gb3, gws, gbs,
            cw1, cb1, cw2, cb2, cws, cbs, z_ref, g_ref):
    y = jnp.concatenate([p_ref[0], p_ref[1], p_ref[2], p_ref[3]], axis=1)

    def mm(a, w, b):
        return jnp.dot(a, w[...], preferred_element_type=jnp.float32) + b[...]

    y2 = mm(_lrelu(mm(y, pw1, pb1)), pw2, pb2)
    h1 = jnp.maximum(mm(y2, gw1, gb1), 0.0)
    h2 = jnp.maximum(mm(h1, gw2, gb2), 0.0)
    h3 = jnp.maximum(mm(h2, gw3, gb3), 0.0)
    g = h3 + mm(y2, gws, gbs)
    hb = _lrelu(mm(_lrelu(mm(g, cw1, cb1)), cw2, cb2))
    z = hb + mm(g, cws, cbs)
    z_ref[...] = z
    g_ref[...] = g


def _heads(pooled, pp, gd, cp):
    emb = NUM_LAYERS * D

    def r(v):
        return v.reshape(1, -1)

    return pl.pallas_call(
        _kheads,
        out_shape=[
            jax.ShapeDtypeStruct((G, cp['Ws'].shape[1]), jnp.float32),
            jax.ShapeDtypeStruct((G, emb), jnp.float32),
        ],
    )(pooled, pp['W1'], r(pp['b1']), pp['W2'], r(pp['b2']),
      gd['W1'], r(gd['b1']), gd['W2'], r(gd['b2']), gd['W3'], r(gd['b3']),
      gd['Ws'], r(gd['bs']),
      cp['W1'], r(cp['b1']), cp['W2'], r(cp['b2']), cp['Ws'], r(cp['bs']))


def kernel(x, edge_index, batch, num_graphs, params):
    src = edge_index[0]
    dst = edge_index[1]
    zrows = jnp.zeros((RP, D), jnp.float32)
    ids = batch.reshape(N, 1)

    h = x
    pooled = []
    for i in range(NUM_LAYERS):
        p = params['gin%d' % i]
        agg = _agg(h, src, dst, zrows)
        u, st = _mlp_stats(h, agg, p['W1'], p['b1'].reshape(1, -1),
                           p['W2'], p['b2'].reshape(1, -1))
        h, pool = _bn_pool(u, st, p['bn_g'].reshape(1, -1),
                           p['bn_b'].reshape(1, -1), ids)
        pooled.append(pool)

    pooled = jnp.stack(pooled)
    z, g = _heads(pooled, params['proj'], params['global_d'],
                  params['cluster'])
    return (z, g)


# speedup vs baseline: 3.6516x; 1.0551x over previous
import functools

import jax
import jax.numpy as jnp
from jax import lax
from jax.experimental import pallas as pl
from jax.experimental.pallas import tpu as pltpu
from jax.experimental.pallas import tpu_sc as plsc

N = 10000
E = 320000
D = 128
G = 128
NUM_LAYERS = 4

NC = 2
NS = 16
NW = NC * NS
K = 80
EPW = E // NW
NB = EPW // K
NPAD = 10240
RP = NPAD // NS

R = 1000
NBLK = N // R


def _make_agg():
    mesh = plsc.VectorSubcoreMesh(core_axis_name="c", subcore_axis_name="s")

    NI = 12
    NR = 4
    GD = 3
    SD = 4

    @functools.partial(
        pl.kernel,
        mesh=mesh,
        out_type=jax.ShapeDtypeStruct((NC, NPAD, D), jnp.float32),
        scratch_types=(
            [pltpu.VMEM((K,), jnp.int32)] * (2 * NI)
            + [pltpu.VMEM((K, D), jnp.float32)] * NR
            + [pltpu.VMEM_SHARED((NPAD, D), jnp.float32)]
            + [pltpu.SemaphoreType.DMA] * (NI + 2 * NR)
        ),
    )
    def agg(h_hbm, src_hbm, dst_hbm, zrows_hbm, out_hbm, *scr):
        sidx = scr[0:NI]
        didx = scr[NI:2 * NI]
        rows = scr[2 * NI:2 * NI + NR]
        shared = scr[2 * NI + NR]
        isem = scr[2 * NI + NR + 1:2 * NI + NR + 1 + NI]
        gsem = scr[2 * NI + NR + 1 + NI:2 * NI + NR + 1 + NI + NR]
        ssem = scr[2 * NI + NR + 1 + NI + NR:]
        c = lax.axis_index("c")
        s = lax.axis_index("s")
        wid = s * NC + c
        r0 = s * RP
        pltpu.sync_copy(zrows_hbm, shared.at[pl.ds(r0, RP)])
        e0 = wid * EPW

        def pref(b, q):
            eb = e0 + b * K
            pltpu.async_copy(src_hbm.at[pl.ds(eb, K)], sidx[q], isem[q])
            pltpu.async_copy(dst_hbm.at[pl.ds(eb, K)], didx[q], isem[q])

        def wait_idx(q):
            pltpu.make_async_copy(src_hbm.at[pl.ds(e0, K)], sidx[q],
                                  isem[q]).wait()
            pltpu.make_async_copy(dst_hbm.at[pl.ds(e0, K)], didx[q],
                                  isem[q]).wait()

        def wait_gather(q6, q4):
            pltpu.make_async_copy(h_hbm.at[sidx[q6]], rows[q4],
                                  gsem[q4]).wait()

        def start_scatter(q6, q4):
            pltpu.async_copy(rows[q4], shared.at[didx[q6]], ssem[q4],
                             add=True)

        def wait_scatter(q6, q4):
            pltpu.make_async_copy(rows[q4], shared.at[didx[q6]],
                                  ssem[q4]).wait()

        pref(0, 0)
        pref(1, 1)
        plsc.subcore_barrier()

        def step(b, q6, q4):
            @pl.when(b >= SD)
            def _():
                wait_scatter((q6 + NI - SD) % NI, q4)

            wait_idx(q6)
            pltpu.async_copy(h_hbm.at[sidx[q6]], rows[q4], gsem[q4])

            @pl.when(b + 2 < NB)
            def _():
                pref(b + 2, (q6 + 2) % NI)

            @pl.when(b >= GD)
            def _():
                wait_gather((q6 + NI - GD) % NI, (q4 + NR - GD) % NR)
                start_scatter((q6 + NI - GD) % NI, (q4 + NR - GD) % NR)

        def body(b, carry):
            for r in range(NI):
                @pl.when(lax.rem(b, NI) == r)
                def _():
                    step(b, r, r % NR)

            return carry

        lax.fori_loop(0, NB, body, 0)

        for bb in range(NB - GD, NB):
            wait_gather(bb % NI, bb % NR)
            start_scatter(bb % NI, bb % NR)
        for bb in range(NB - SD, NB):
            wait_scatter(bb % NI, bb % NR)

        plsc.subcore_barrier()
        pltpu.sync_copy(shared.at[pl.ds(r0, RP)], out_hbm.at[c, pl.ds(r0, RP)])

    return agg


_agg_cache = []


def _agg(h, src, dst, zrows):
    if not _agg_cache:
        _agg_cache.append(_make_agg())
    return _agg_cache[0](h, src, dst, zrows)


def _k1(h_ref, a_ref, w1_ref, b1_ref, w2_ref, b2_ref, u_ref, st_ref, acc_ref):
    z = h_ref[...] + a_ref[0] + a_ref[1]
    t = jnp.dot(z, w1_ref[...], preferred_element_type=jnp.float32) + b1_ref[...]
    t = jnp.maximum(t, 0.0)
    u = jnp.dot(t, w2_ref[...], preferred_element_type=jnp.float32) + b2_ref[...]
    u = jnp.maximum(u, 0.0)
    u_ref[...] = u
    ssum = jnp.sum(u, axis=0, keepdims=True)
    ssq = jnp.sum(u * u, axis=0, keepdims=True)
    stats = jnp.concatenate([ssum, ssq], axis=0)

    @pl.when(pl.program_id(0) == 0)
    def _():
        acc_ref[...] = jnp.zeros_like(acc_ref)

    acc_ref[...] += stats

    @pl.when(pl.program_id(0) == pl.num_programs(0) - 1)
    def _():
        st_ref[...] = acc_ref[...]


def _mlp_stats(h, agg, w1, b1, w2, b2):
    return pl.pallas_call(
        _k1,
        grid=(NBLK,),
        in_specs=[
            pl.BlockSpec((R, D), lambda i: (i, 0)),
            pl.BlockSpec((NC, R, D), lambda i: (0, i, 0)),
            pl.BlockSpec((D, D), lambda i: (0, 0)),
            pl.BlockSpec((1, D), lambda i: (0, 0)),
            pl.BlockSpec((D, D), lambda i: (0, 0)),
            pl.BlockSpec((1, D), lambda i: (0, 0)),
        ],
        out_specs=[
            pl.BlockSpec((R, D), lambda i: (i, 0)),
            pl.BlockSpec((2, D), lambda i: (0, 0)),
        ],
        out_shape=[
            jax.ShapeDtypeStruct((N, D), jnp.float32),
            jax.ShapeDtypeStruct((2, D), jnp.float32),
        ],
        scratch_shapes=[pltpu.VMEM((2, D), jnp.float32)],
    )(h, agg, w1, b1, w2, b2)


def _k2(u_ref, st_ref, g_ref, bt_ref, ids_ref, h_ref, p_ref, acc_ref):
    mean = st_ref[0:1] * (1.0 / N)
    var = st_ref[1:2] * (1.0 / N) - mean * mean
    inv = g_ref[...] * lax.rsqrt(var + 1e-5)
    h = (u_ref[...] - mean) * inv + bt_ref[...]
    h_ref[...] = h
    oh = (ids_ref[...] == lax.broadcasted_iota(jnp.int32, (1, G), 1))
    oh = oh.astype(jnp.float32)
    p = lax.dot_general(oh, h, (((0,), (0,)), ((), ())),
                        precision=lax.Precision.HIGHEST,
                        preferred_element_type=jnp.float32)

    @pl.when(pl.program_id(0) == 0)
    def _():
        acc_ref[...] = jnp.zeros_like(acc_ref)

    acc_ref[...] += p

    @pl.when(pl.program_id(0) == pl.num_programs(0) - 1)
    def _():
        p_ref[...] = acc_ref[...]


def _bn_pool(u, st, gam, bet, ids):
    return pl.pallas_call(
        _k2,
        grid=(NBLK,),
        in_specs=[
            pl.BlockSpec((R, D), lambda i: (i, 0)),
            pl.BlockSpec((2, D), lambda i: (0, 0)),
            pl.BlockSpec((1, D), lambda i: (0, 0)),
            pl.BlockSpec((1, D), lambda i: (0, 0)),
            pl.BlockSpec((R, 1), lambda i: (i, 0)),
        ],
        out_specs=[
            pl.BlockSpec((R, D), lambda i: (i, 0)),
            pl.BlockSpec((G, D), lambda i: (0, 0)),
        ],
        out_shape=[
            jax.ShapeDtypeStruct((N, D), jnp.float32),
            jax.ShapeDtypeStruct((G, D), jnp.float32),
        ],
        scratch_shapes=[pltpu.VMEM((G, D), jnp.float32)],
    )(u, st, gam, bet, ids)


def _lrelu(x):
    return jnp.where(x >= 0, x, 0.01 * x)


def _kheads(p_ref, pw1, pb1, pw2, pb2, gw1, gb1, gw2, gb2, gw3, gb3, gws, gbs,
            cw1, cb1, cw2, cb2, cws, cbs, z_ref, g_ref):
    y = jnp.concatenate([p_ref[0], p_ref[1], p_ref[2], p_ref[3]], axis=1)

    def mm(a, w, b):
        return jnp.dot(a, w[...], preferred_element_type=jnp.float32) + b[...]

    y2 = mm(_lrelu(mm(y, pw1, pb1)), pw2, pb2)
    h1 = jnp.maximum(mm(y2, gw1, gb1), 0.0)
    h2 = jnp.maximum(mm(h1, gw2, gb2), 0.0)
    h3 = jnp.maximum(mm(h2, gw3, gb3), 0.0)
    g = h3 + mm(y2, gws, gbs)
    hb = _lrelu(mm(_lrelu(mm(g, cw1, cb1)), cw2, cb2))
    z = hb + mm(g, cws, cbs)
    z_ref[...] = z
    g_ref[...] = g


def _heads(pooled, pp, gd, cp):
    emb = NUM_LAYERS * D

    def r(v):
        return v.reshape(1, -1)

    return pl.pallas_call(
        _kheads,
        out_shape=[
            jax.ShapeDtypeStruct((G, cp['Ws'].shape[1]), jnp.float32),
            jax.ShapeDtypeStruct((G, emb), jnp.float32),
        ],
    )(pooled, pp['W1'], r(pp['b1']), pp['W2'], r(pp['b2']),
      gd['W1'], r(gd['b1']), gd['W2'], r(gd['b2']), gd['W3'], r(gd['b3']),
      gd['Ws'], r(gd['bs']),
      cp['W1'], r(cp['b1']), cp['W2'], r(cp['b2']), cp['Ws'], r(cp['bs']))


def kernel(x, edge_index, batch, num_graphs, params):
    src = edge_index[0]
    dst = edge_index[1]
    zrows = jnp.zeros((RP, D), jnp.float32)
    ids = batch.reshape(N, 1)

    h = x
    pooled = []
    for i in range(NUM_LAYERS):
        p = params['gin%d' % i]
        agg = _agg(h, src, dst, zrows)
        u, st = _mlp_stats(h, agg, p['W1'], p['b1'].reshape(1, -1),
                           p['W2'], p['b2'].reshape(1, -1))
        h, pool = _bn_pool(u, st, p['bn_g'].reshape(1, -1),
                           p['bn_b'].reshape(1, -1), ids)
        pooled.append(pool)

    pooled = jnp.stack(pooled)
    z, g = _heads(pooled, params['proj'], params['global_d'],
                  params['cluster'])
    return (z, g)


# speedup vs baseline: 3.7671x; 1.0316x over previous
import functools

import jax
import jax.numpy as jnp
from jax import lax
from jax.experimental import pallas as pl
from jax.experimental.pallas import tpu as pltpu
from jax.experimental.pallas import tpu_sc as plsc

N = 10000
E = 320000
D = 128
G = 128
NUM_LAYERS = 4

NC = 2
NS = 16
NW = NC * NS
K = 80
EPW = E // NW
NB = EPW // K
NPAD = 10240
RP = NPAD // NS

R = 1000
NBLK = N // R


def _make_agg():
    mesh = plsc.VectorSubcoreMesh(core_axis_name="c", subcore_axis_name="s")

    NI = 12
    NR = 4
    GD = 3
    SD = 4

    @functools.partial(
        pl.kernel,
        mesh=mesh,
        out_type=jax.ShapeDtypeStruct((NC, NPAD, D), jnp.float32),
        scratch_types=(
            [pltpu.VMEM((K,), jnp.int32)] * (2 * NI)
            + [pltpu.VMEM((K, D), jnp.float32)] * NR
            + [pltpu.VMEM_SHARED((NPAD, D), jnp.float32)]
            + [pltpu.SemaphoreType.DMA] * (NI + 2 * NR)
        ),
    )
    def agg(h_hbm, src_hbm, dst_hbm, zrows_hbm, out_hbm, *scr):
        sidx = scr[0:NI]
        didx = scr[NI:2 * NI]
        rows = scr[2 * NI:2 * NI + NR]
        shared = scr[2 * NI + NR]
        isem = scr[2 * NI + NR + 1:2 * NI + NR + 1 + NI]
        gsem = scr[2 * NI + NR + 1 + NI:2 * NI + NR + 1 + NI + NR]
        ssem = scr[2 * NI + NR + 1 + NI + NR:]
        c = lax.axis_index("c")
        s = lax.axis_index("s")
        wid = s * NC + c
        r0 = s * RP
        pltpu.sync_copy(zrows_hbm, shared.at[pl.ds(r0, RP)])
        e0 = wid * EPW

        def pref(b, q):
            eb = e0 + b * K
            pltpu.async_copy(src_hbm.at[pl.ds(eb, K)], sidx[q], isem[q])
            pltpu.async_copy(dst_hbm.at[pl.ds(eb, K)], didx[q], isem[q])

        def wait_idx(q):
            pltpu.make_async_copy(src_hbm.at[pl.ds(e0, K)], sidx[q],
                                  isem[q]).wait()
            pltpu.make_async_copy(dst_hbm.at[pl.ds(e0, K)], didx[q],
                                  isem[q]).wait()

        def wait_gather(q6, q4):
            pltpu.make_async_copy(h_hbm.at[sidx[q6]], rows[q4],
                                  gsem[q4]).wait()

        def start_scatter(q6, q4):
            pltpu.async_copy(rows[q4], shared.at[didx[q6]], ssem[q4],
                             add=True)

        def wait_scatter(q6, q4):
            pltpu.make_async_copy(rows[q4], shared.at[didx[q6]],
                                  ssem[q4]).wait()

        pref(0, 0)
        pref(1, 1)
        plsc.subcore_barrier()

        def step(b, q6, q4):
            @pl.when(b >= SD)
            def _():
                wait_scatter((q6 + NI - SD) % NI, q4)

            wait_idx(q6)
            pltpu.async_copy(h_hbm.at[sidx[q6]], rows[q4], gsem[q4])

            @pl.when(b + 2 < NB)
            def _():
                pref(b + 2, (q6 + 2) % NI)

            @pl.when(b >= GD)
            def _():
                wait_gather((q6 + NI - GD) % NI, (q4 + NR - GD) % NR)
                start_scatter((q6 + NI - GD) % NI, (q4 + NR - GD) % NR)

        def body(b, carry):
            for r in range(NI):
                @pl.when(lax.rem(b, NI) == r)
                def _():
                    step(b, r, r % NR)

            return carry

        lax.fori_loop(0, NB, body, 0)

        for bb in range(NB - GD, NB):
            wait_gather(bb % NI, bb % NR)
            start_scatter(bb % NI, bb % NR)
        for bb in range(NB - SD, NB):
            wait_scatter(bb % NI, bb % NR)

        plsc.subcore_barrier()
        pltpu.sync_copy(shared.at[pl.ds(r0, RP)], out_hbm.at[c, pl.ds(r0, RP)])

    return agg


_agg_cache = []


def _agg(h, src, dst, zrows):
    if not _agg_cache:
        _agg_cache.append(_make_agg())
    return _agg_cache[0](h, src, dst, zrows)


def _k12(h_ref, a_ref, w1_ref, b1_ref, w2_ref, b2_ref, g_ref, bt_ref,
         ids_ref, hout_ref, pool_ref, u_scr, st_ref, pacc_ref):
    ph = pl.program_id(0)
    i = pl.program_id(1)

    @pl.when(ph == 0)
    def _():
        z = h_ref[...] + a_ref[0] + a_ref[1]
        t = jnp.dot(z, w1_ref[...],
                    preferred_element_type=jnp.float32) + b1_ref[...]
        t = jnp.maximum(t, 0.0)
        u = jnp.dot(t, w2_ref[...],
                    preferred_element_type=jnp.float32) + b2_ref[...]
        u = jnp.maximum(u, 0.0)
        u_scr[pl.ds(i * R, R), :] = u
        ssum = jnp.sum(u, axis=0, keepdims=True)
        ssq = jnp.sum(u * u, axis=0, keepdims=True)
        stats = jnp.concatenate([ssum, ssq], axis=0)

        @pl.when(i == 0)
        def _():
            st_ref[...] = jnp.zeros_like(st_ref)

        st_ref[...] += stats

    @pl.when(ph == 1)
    def _():
        u = u_scr[pl.ds(i * R, R), :]
        mean = st_ref[0:1] * (1.0 / N)
        var = st_ref[1:2] * (1.0 / N) - mean * mean
        inv = g_ref[...] * lax.rsqrt(var + 1e-5)
        h = (u - mean) * inv + bt_ref[...]
        hout_ref[...] = h
        oh = (ids_ref[...] == lax.broadcasted_iota(jnp.int32, (1, G), 1))
        oh = oh.astype(jnp.float32)
        p = lax.dot_general(oh, h, (((0,), (0,)), ((), ())),
                            precision=lax.Precision.HIGHEST,
                            preferred_element_type=jnp.float32)

        @pl.when(i == 0)
        def _():
            pacc_ref[...] = jnp.zeros_like(pacc_ref)

        pacc_ref[...] += p

        @pl.when(i == pl.num_programs(1) - 1)
        def _():
            pool_ref[...] = pacc_ref[...]


def _layer_tc(h, agg, w1, b1, w2, b2, gam, bet, ids):
    return pl.pallas_call(
        _k12,
        grid=(2, NBLK),
        in_specs=[
            pl.BlockSpec((R, D), lambda p, i: ((1 - p) * i, 0)),
            pl.BlockSpec((NC, R, D), lambda p, i: (0, (1 - p) * i, 0)),
            pl.BlockSpec((D, D), lambda p, i: (0, 0)),
            pl.BlockSpec((1, D), lambda p, i: (0, 0)),
            pl.BlockSpec((D, D), lambda p, i: (0, 0)),
            pl.BlockSpec((1, D), lambda p, i: (0, 0)),
            pl.BlockSpec((1, D), lambda p, i: (0, 0)),
            pl.BlockSpec((1, D), lambda p, i: (0, 0)),
            pl.BlockSpec((R, 1), lambda p, i: (p * i, 0)),
        ],
        out_specs=[
            pl.BlockSpec((R, D), lambda p, i: (p * i, 0)),
            pl.BlockSpec((G, D), lambda p, i: (0, 0)),
        ],
        out_shape=[
            jax.ShapeDtypeStruct((N, D), jnp.float32),
            jax.ShapeDtypeStruct((G, D), jnp.float32),
        ],
        scratch_shapes=[
            pltpu.VMEM((N, D), jnp.float32),
            pltpu.VMEM((2, D), jnp.float32),
            pltpu.VMEM((G, D), jnp.float32),
        ],
    )(h, agg, w1, b1, w2, b2, gam, bet, ids)


def _lrelu(x):
    return jnp.where(x >= 0, x, 0.01 * x)


def _kheads(p_ref, pw1, pb1, pw2, pb2, gw1, gb1, gw2, gb2, gw3, gb3, gws, gbs,
            cw1, cb1, cw2, cb2, cws, cbs, z_ref, g_ref):
    y = jnp.concatenate([p_ref[0], p_ref[1], p_ref[2], p_ref[3]], axis=1)

    def mm(a, w, b):
        return jnp.dot(a, w[...], preferred_element_type=jnp.float32) + b[...]

    y2 = mm(_lrelu(mm(y, pw1, pb1)), pw2, pb2)
    h1 = jnp.maximum(mm(y2, gw1, gb1), 0.0)
    h2 = jnp.maximum(mm(h1, gw2, gb2), 0.0)
    h3 = jnp.maximum(mm(h2, gw3, gb3), 0.0)
    g = h3 + mm(y2, gws, gbs)
    hb = _lrelu(mm(_lrelu(mm(g, cw1, cb1)), cw2, cb2))
    z = hb + mm(g, cws, cbs)
    z_ref[...] = z
    g_ref[...] = g


def _heads(pooled, pp, gd, cp):
    emb = NUM_LAYERS * D

    def r(v):
        return v.reshape(1, -1)

    return pl.pallas_call(
        _kheads,
        out_shape=[
            jax.ShapeDtypeStruct((G, cp['Ws'].shape[1]), jnp.float32),
            jax.ShapeDtypeStruct((G, emb), jnp.float32),
        ],
    )(pooled, pp['W1'], r(pp['b1']), pp['W2'], r(pp['b2']),
      gd['W1'], r(gd['b1']), gd['W2'], r(gd['b2']), gd['W3'], r(gd['b3']),
      gd['Ws'], r(gd['bs']),
      cp['W1'], r(cp['b1']), cp['W2'], r(cp['b2']), cp['Ws'], r(cp['bs']))


def kernel(x, edge_index, batch, num_graphs, params):
    src = edge_index[0]
    dst = edge_index[1]
    zrows = jnp.zeros((RP, D), jnp.float32)
    ids = batch.reshape(N, 1)

    h = x
    pooled = []
    for i in range(NUM_LAYERS):
        p = params['gin%d' % i]
        agg = _agg(h, src, dst, zrows)
        h, pool = _layer_tc(h, agg, p['W1'], p['b1'].reshape(1, -1),
                            p['W2'], p['b2'].reshape(1, -1),
                            p['bn_g'].reshape(1, -1),
                            p['bn_b'].reshape(1, -1), ids)
        pooled.append(pool)

    pooled = jnp.stack(pooled)
    z, g = _heads(pooled, params['proj'], params['global_d'],
                  params['cluster'])
    return (z, g)
